# Initial kernel scaffold; baseline (speedup 1.0000x reference)
import jax
import jax.numpy as jnp
from jax.experimental import pallas as pl


def kernel(src, tar):
    raise NotImplementedError("write your pallas kernel here")


# speedup vs baseline: 1.5351x; 1.5351x over previous
import jax
import jax.numpy as jnp
from jax.experimental import pallas as pl

_BQ = 1024


def _argmin_body(q_ref, rt_ref, out_ref):
    q = q_ref[0]
    rt = rt_ref[0]
    n = rt.shape[1]
    c = jnp.sum(rt * rt, axis=0, keepdims=True)
    t = c - 2.0 * jnp.dot(q, rt, preferred_element_type=jnp.float32)
    mv = jnp.min(t, axis=1, keepdims=True)
    iota = jax.lax.broadcasted_iota(jnp.int32, t.shape, 1)
    idx = jnp.min(jnp.where(t == mv, iota, n), axis=1)
    out_ref[0, 0, :] = idx


def kernel(src, tar):
    B, N, _ = src.shape
    q = jnp.concatenate([src, tar], axis=0)
    r = jnp.concatenate([tar, src], axis=0)
    qp = jnp.pad(q, ((0, 0), (0, 0), (0, 5)))
    rtp = jnp.pad(r.transpose(0, 2, 1), ((0, 0), (0, 5), (0, 0)))

    idx = pl.pallas_call(
        _argmin_body,
        grid=(2 * B, N // _BQ),
        in_specs=[
            pl.BlockSpec((1, _BQ, 8), lambda b, t: (b, t, 0)),
            pl.BlockSpec((1, 8, N), lambda b, t: (b, 0, 0)),
        ],
        out_specs=pl.BlockSpec((1, 1, _BQ), lambda b, t: (b * (N // _BQ) + t, 0, 0)),
        out_shape=jax.ShapeDtypeStruct((2 * B * (N // _BQ), 1, _BQ), jnp.int32),
    )(qp, rtp)
    idx = idx.reshape(2 * B, N)

    nn = jnp.take_along_axis(r, idx[:, :, None], axis=1)
    diff = nn - q
    dist = jnp.sqrt(jnp.sum(diff * diff, axis=-1))
    acc = jnp.mean(dist[:B], axis=1)
    com = jnp.mean(dist[B:], axis=1)
    return 0.5 * (acc + com)
